# baseline (speedup 1.0000x reference)
import functools

import jax
import jax.numpy as jnp
from jax import lax
from jax.experimental import pallas as pl
from jax.experimental.pallas import tpu as pltpu
from jax.experimental.pallas import tpu_sc as plsc

SEQ = 50
D = 32
VOCAB = 1000000
BATCH = 16384
NCLS = 13
TOTAL = BATCH * SEQ

_NC, _NS = 2, 16
NW = _NC * _NS
PER_W = TOTAL // NW
CHUNK = 3200
N_CHUNKS = PER_W // CHUNK

BK = 2048
NB = -(-VOCAB // BK)
LIN_ROWS = VOCAB * D // 128


def _detile(tt):
    def body(t_ref, o_ref, s_ref):
        i = pl.program_id(0)

        @pl.when(i == 0)
        def _():
            r = lax.broadcasted_iota(jnp.int32, (4, BK // 4, BK), 1)
            c = lax.broadcasted_iota(jnp.int32, (4, BK // 4, BK), 2)
            g = lax.broadcasted_iota(jnp.int32, (4, BK // 4, BK), 0)
            s_ref[...] = (c == 4 * r + g).astype(jnp.float32)

        t = t_ref[...]
        for g in range(4):
            p = lax.dot_general(
                s_ref[g], t, (((1,), (1,)), ((), ())),
                preferred_element_type=jnp.float32,
            )
            o_ref[:, 32 * g : 32 * (g + 1)] = p

    return pl.pallas_call(
        body,
        grid=(NB,),
        in_specs=[pl.BlockSpec((D, BK), lambda i: (0, i))],
        out_specs=pl.BlockSpec((BK // 4, 128), lambda i: (i, 0)),
        out_shape=jax.ShapeDtypeStruct((LIN_ROWS, 128), jnp.float32),
        scratch_shapes=[pltpu.VMEM((4, BK // 4, BK), jnp.float32)],
    )(tt)


def _gather(xf, table):
    mesh = plsc.VectorSubcoreMesh(core_axis_name="c", subcore_axis_name="s")

    @functools.partial(
        pl.kernel,
        mesh=mesh,
        out_type=jax.ShapeDtypeStruct((TOTAL, D), jnp.float32),
        scratch_types=[
            pltpu.VMEM((CHUNK,), jnp.int32),
            pltpu.VMEM((CHUNK, D), jnp.float32),
            pltpu.SemaphoreType.DMA,
        ],
        compiler_params=pltpu.CompilerParams(use_tc_tiling_on_sc=False),
    )
    def k(x_hbm, table_hbm, out_hbm, idx_v, rows_v, sem):
        wid = lax.axis_index("s") * _NC + lax.axis_index("c")
        base = wid * PER_W

        def body(i, carry):
            off = pl.multiple_of(base + i * CHUNK, CHUNK)
            pltpu.sync_copy(x_hbm.at[pl.ds(off, CHUNK)], idx_v)
            pltpu.async_copy(table_hbm.at[idx_v], rows_v, sem).wait()
            pltpu.sync_copy(rows_v, out_hbm.at[pl.ds(off, CHUNK)])
            return carry

        lax.fori_loop(0, N_CHUNKS, body, 0)

    return k(xf, table)


def _linear(z5, wbig):
    def body(z_ref, w_ref, o_ref):
        l = pl.program_id(0)
        p = jnp.dot(z_ref[0], w_ref[0], preferred_element_type=jnp.float32)

        @pl.when(l == 0)
        def _():
            o_ref[...] = p

        @pl.when(l > 0)
        def _():
            o_ref[...] += p

    return pl.pallas_call(
        body,
        grid=(SEQ,),
        in_specs=[
            pl.BlockSpec((1, BATCH // 4, 128), lambda l: (l, 0, 0)),
            pl.BlockSpec((1, 128, 4 * NCLS), lambda l: (l, 0, 0)),
        ],
        out_specs=pl.BlockSpec((BATCH // 4, 4 * NCLS), lambda l: (0, 0)),
        out_shape=jax.ShapeDtypeStruct((BATCH // 4, 4 * NCLS), jnp.float32),
    )(z5, wbig)


def kernel(x, table, W, b):
    xf = x.T.reshape(-1).astype(jnp.int32)
    tlin = _detile(table.T)
    tbl = tlin.reshape(VOCAB, D)
    rows = _gather(xf, tbl)
    z5 = rows.reshape(SEQ, BATCH // 4, 128)
    wr = W.reshape(NCLS, SEQ, D).transpose(1, 2, 0)
    wbig = jnp.einsum("ldc,gh->lgdhc", wr, jnp.eye(4, dtype=W.dtype))
    wbig = wbig.reshape(SEQ, 128, 4 * NCLS)
    acc = _linear(z5, wbig)
    out = acc.reshape(BATCH // 4, 4, NCLS).reshape(BATCH, NCLS)
    return out + b[None, :]


# speedup vs baseline: 2.0469x; 2.0469x over previous
import functools

import jax
import jax.numpy as jnp
from jax import lax
from jax.experimental import pallas as pl
from jax.experimental.pallas import tpu as pltpu
from jax.experimental.pallas import tpu_sc as plsc

SEQ = 50
D = 32
VOCAB = 1000000
BATCH = 16384
NCLS = 13
TOTAL = BATCH * SEQ

_NC, _NS = 2, 16
NW = _NC * _NS
PER_W = TOTAL // NW
CHUNK = 3200
N_CHUNKS = PER_W // CHUNK

BK = 2048
NB = -(-VOCAB // BK)
LIN_ROWS = VOCAB * D // 128


def _detile(tt):
    def body(t_ref, o_ref):
        a = t_ref[...].T
        a3 = a.reshape(BK // 4, 4, 32)
        acc = None
        for g in range(4):
            ag = a3[:, g, :]
            pg = jnp.pad(ag, ((0, 0), (32 * g, 96 - 32 * g)))
            acc = pg if acc is None else acc + pg
        o_ref[...] = acc

    return pl.pallas_call(
        body,
        grid=(NB,),
        in_specs=[pl.BlockSpec((D, BK), lambda i: (0, i))],
        out_specs=pl.BlockSpec((BK // 4, 128), lambda i: (i, 0)),
        out_shape=jax.ShapeDtypeStruct((LIN_ROWS, 128), jnp.float32),
    )(tt)


def _gather(xf, table):
    mesh = plsc.VectorSubcoreMesh(core_axis_name="c", subcore_axis_name="s")

    @functools.partial(
        pl.kernel,
        mesh=mesh,
        out_type=jax.ShapeDtypeStruct((TOTAL, D), jnp.float32),
        scratch_types=[
            pltpu.VMEM((CHUNK,), jnp.int32),
            pltpu.VMEM((CHUNK, D), jnp.float32),
            pltpu.SemaphoreType.DMA,
        ],
        compiler_params=pltpu.CompilerParams(use_tc_tiling_on_sc=False),
    )
    def k(x_hbm, table_hbm, out_hbm, idx_v, rows_v, sem):
        wid = lax.axis_index("s") * _NC + lax.axis_index("c")
        base = wid * PER_W

        def body(i, carry):
            off = pl.multiple_of(base + i * CHUNK, CHUNK)
            pltpu.sync_copy(x_hbm.at[pl.ds(off, CHUNK)], idx_v)
            pltpu.async_copy(table_hbm.at[idx_v], rows_v, sem).wait()
            pltpu.sync_copy(rows_v, out_hbm.at[pl.ds(off, CHUNK)])
            return carry

        lax.fori_loop(0, N_CHUNKS, body, 0)

    return k(xf, table)


def _linear(z5, wbig):
    def body(z_ref, w_ref, o_ref):
        l = pl.program_id(0)
        p = jnp.dot(z_ref[0], w_ref[0], preferred_element_type=jnp.float32)

        @pl.when(l == 0)
        def _():
            o_ref[...] = p

        @pl.when(l > 0)
        def _():
            o_ref[...] += p

    return pl.pallas_call(
        body,
        grid=(SEQ,),
        in_specs=[
            pl.BlockSpec((1, BATCH // 4, 128), lambda l: (l, 0, 0)),
            pl.BlockSpec((1, 128, 4 * NCLS), lambda l: (l, 0, 0)),
        ],
        out_specs=pl.BlockSpec((BATCH // 4, 4 * NCLS), lambda l: (0, 0)),
        out_shape=jax.ShapeDtypeStruct((BATCH // 4, 4 * NCLS), jnp.float32),
    )(z5, wbig)


def kernel(x, table, W, b):
    xf = x.T.reshape(-1).astype(jnp.int32)
    tlin = _detile(table.T)
    tbl = tlin.reshape(VOCAB, D)
    rows = _gather(xf, tbl)
    z5 = rows.reshape(SEQ, BATCH // 4, 128)
    wr = W.reshape(NCLS, SEQ, D).transpose(1, 2, 0)
    wbig = jnp.einsum("ldc,gh->lgdhc", wr, jnp.eye(4, dtype=W.dtype))
    wbig = wbig.reshape(SEQ, 128, 4 * NCLS)
    acc = _linear(z5, wbig)
    out = acc.reshape(BATCH // 4, 4, NCLS).reshape(BATCH, NCLS)
    return out + b[None, :]
